# baseline (speedup 1.0000x reference)
import functools

import jax
import jax.numpy as jnp
from jax import lax
from jax.experimental import pallas as pl
from jax.experimental.pallas import tpu as pltpu
from jax.experimental.pallas import tpu_sc as plsc


def _make_table_pad(v, d_model, d_pad, vblk):
    grid = -(-v // vblk)

    def body(src_ref, dst_ref):
        dst_ref[:, 0:d_model] = src_ref[...].T

    return pl.pallas_call(
        body,
        grid=(grid,),
        in_specs=[pl.BlockSpec((d_model, vblk), lambda j: (0, j))],
        out_specs=pl.BlockSpec((vblk, d_pad), lambda j: (j, 0)),
        out_shape=jax.ShapeDtypeStruct((v, d_pad), jnp.float32),
    )


def _make_lookup(seq, batch, d_model, d_pad, blk):
    info = plsc.get_sparse_core_info()
    nc, ns, lanes = info.num_cores, info.num_subcores, info.num_lanes
    nw = nc * ns
    assert batch == nw * blk
    assert blk % lanes == 0
    assert seq % 2 == 0

    mesh = plsc.VectorSubcoreMesh(core_axis_name="c", subcore_axis_name="s")

    @functools.partial(
        pl.kernel,
        mesh=mesh,
        compiler_params=pltpu.CompilerParams(needs_layout_passes=False),
        out_type=jax.ShapeDtypeStruct((seq, d_model, batch), jnp.float32),
        scratch_types=[
            pltpu.VMEM((seq, blk), jnp.int32),
            pltpu.VMEM((blk, d_pad), jnp.float32),
            pltpu.VMEM((blk, d_pad), jnp.float32),
            pltpu.VMEM((d_model, blk + 1), jnp.float32),
            pltpu.VMEM((d_model, blk + 1), jnp.float32),
            pltpu.SemaphoreType.DMA,
            pltpu.SemaphoreType.DMA,
            pltpu.SemaphoreType.DMA,
            pltpu.SemaphoreType.DMA,
        ],
    )
    def k(idx_hbm, table_hbm, out_hbm, idx_v, gbuf0, gbuf1, tbuf0, tbuf1,
          gsem0, gsem1, osem0, osem1):
        wid = lax.axis_index("s") * nc + lax.axis_index("c")
        b0 = wid * blk
        dm_tab = [lax.iota(jnp.int32, lanes) + (db * lanes)
                  for db in range(d_model // lanes)]

        pltpu.sync_copy(idx_hbm.at[:, pl.ds(b0, blk)], idx_v)
        pltpu.async_copy(table_hbm.at[idx_v.at[0]], gbuf0, gsem0)

        def transpose(gbuf, tbuf):
            @plsc.parallel_loop(0, blk, step=1, unroll=8)
            def _(j):
                colj = jnp.full((lanes,), j, jnp.int32)
                for db in range(d_model // lanes):
                    vals = gbuf[j, pl.ds(db * lanes, lanes)]
                    plsc.store_scatter(tbuf, [dm_tab[db], colj], vals)

        def phase(s, nxt_ok, gbuf, gsem, gbuf_n, gsem_n, tbuf, osem, warm):
            @pl.when(nxt_ok)
            def _():
                pltpu.async_copy(table_hbm.at[idx_v.at[s + 1]], gbuf_n, gsem_n)

            pltpu.make_async_copy(table_hbm.at[idx_v.at[0]], gbuf, gsem).wait()

            @pl.when(warm)
            def _():
                pltpu.make_async_copy(
                    tbuf.at[:, pl.ds(0, blk)], out_hbm.at[0, :, pl.ds(b0, blk)],
                    osem
                ).wait()

            transpose(gbuf, tbuf)
            pltpu.async_copy(tbuf.at[:, pl.ds(0, blk)],
                             out_hbm.at[s, :, pl.ds(b0, blk)], osem)

        def body(g, carry):
            s = g * 2
            phase(s, s + 1 < seq, gbuf0, gsem0, gbuf1, gsem1, tbuf0, osem0,
                  g > 0)
            phase(s + 1, s + 2 < seq, gbuf1, gsem1, gbuf0, gsem0, tbuf1, osem1,
                  g > 0)
            return carry

        lax.fori_loop(0, seq // 2, body, 0)
        pltpu.make_async_copy(tbuf0.at[:, pl.ds(0, blk)],
                              out_hbm.at[0, :, pl.ds(b0, blk)], osem0).wait()
        pltpu.make_async_copy(tbuf1.at[:, pl.ds(0, blk)],
                              out_hbm.at[0, :, pl.ds(b0, blk)], osem1).wait()

    return k


def kernel(x, embs):
    b, s = x.shape
    v, d = embs.shape
    d_pad = 128
    idx_t = x.T.astype(jnp.int32)
    table = _make_table_pad(v, d, d_pad, 4096)(embs.T)
    out_t = _make_lookup(s, b, d, d_pad, 128)(idx_t, table)
    return jnp.transpose(out_t, (2, 0, 1))


# speedup vs baseline: 1.0696x; 1.0696x over previous
import functools

import jax
import jax.numpy as jnp
from jax import lax
from jax.experimental import pallas as pl
from jax.experimental.pallas import tpu as pltpu
from jax.experimental.pallas import tpu_sc as plsc


def _make_table_pair(v, d_model, vblk):
    grid = -(-v // vblk)
    d2 = 2 * d_model

    def body(src_ref, dst_ref):
        dst_ref[:, 0:d_model] = src_ref[:, 0:vblk // 2].T
        dst_ref[:, d_model:d2] = src_ref[:, vblk // 2:vblk].T

    return pl.pallas_call(
        body,
        grid=(grid,),
        in_specs=[pl.BlockSpec((d_model, vblk), lambda j: (0, j))],
        out_specs=pl.BlockSpec((vblk // 2, d2), lambda j: (j, 0)),
        out_shape=jax.ShapeDtypeStruct((v // 2, d2), jnp.float32),
    )


def _make_lookup(seq, batch, d_model, d_pad, blk):
    info = plsc.get_sparse_core_info()
    nc, ns, lanes = info.num_cores, info.num_subcores, info.num_lanes
    nw = nc * ns
    assert batch == nw * blk
    assert blk % lanes == 0
    assert seq % 2 == 0

    mesh = plsc.VectorSubcoreMesh(core_axis_name="c", subcore_axis_name="s")

    @functools.partial(
        pl.kernel,
        mesh=mesh,
        compiler_params=pltpu.CompilerParams(needs_layout_passes=False),
        out_type=jax.ShapeDtypeStruct((seq, d_model, batch), jnp.float32),
        scratch_types=[
            pltpu.VMEM((seq, blk), jnp.int32),
            pltpu.VMEM((seq, blk), jnp.int32),
            pltpu.VMEM((blk, d_pad), jnp.float32),
            pltpu.VMEM((blk, d_pad), jnp.float32),
            pltpu.VMEM((d_model, blk), jnp.float32),
            pltpu.VMEM((d_model, blk), jnp.float32),
            pltpu.SemaphoreType.DMA,
            pltpu.SemaphoreType.DMA,
            pltpu.SemaphoreType.DMA,
            pltpu.SemaphoreType.DMA,
        ],
    )
    def k(idx_hbm, table_hbm, out_hbm, idx_v, par_v, gbuf0, gbuf1, tbuf0,
          tbuf1, gsem0, gsem1, osem0, osem1):
        wid = lax.axis_index("s") * nc + lax.axis_index("c")
        b0 = wid * blk
        rows_tab = [lax.iota(jnp.int32, lanes) + (jb * lanes)
                    for jb in range(blk // lanes)]

        pltpu.sync_copy(idx_hbm.at[:, pl.ds(b0, blk)], idx_v)

        @plsc.parallel_loop(0, seq, step=1, unroll=4)
        def _(s):
            for jb in range(blk // lanes):
                raw = idx_v[s, pl.ds(jb * lanes, lanes)]
                par_v[s, pl.ds(jb * lanes, lanes)] = (
                    (raw >> 11) & 1) * d_model
                idx_v[s, pl.ds(jb * lanes, lanes)] = (
                    ((raw >> 12) << 11) | (raw & 2047))

        pltpu.async_copy(table_hbm.at[idx_v.at[0]], gbuf0, gsem0)

        def transpose(gbuf, par_row, tbuf):
            par_vecs = [par_row[pl.ds(jb * lanes, lanes)]
                        for jb in range(blk // lanes)]

            @plsc.parallel_loop(0, d_model, step=1, unroll=8)
            def _(dm):
                for jb in range(blk // lanes):
                    tbuf[dm, pl.ds(jb * lanes, lanes)] = plsc.load_gather(
                        gbuf, [rows_tab[jb], par_vecs[jb] + dm]
                    )

        def phase(s, nxt_ok, gbuf, gsem, gbuf_n, gsem_n, tbuf, osem, warm):
            @pl.when(nxt_ok)
            def _():
                pltpu.async_copy(table_hbm.at[idx_v.at[s + 1]], gbuf_n, gsem_n)

            pltpu.make_async_copy(table_hbm.at[idx_v.at[0]], gbuf, gsem).wait()

            @pl.when(warm)
            def _():
                pltpu.make_async_copy(
                    tbuf, out_hbm.at[0, :, pl.ds(b0, blk)], osem
                ).wait()

            transpose(gbuf, par_v.at[s], tbuf)
            pltpu.async_copy(tbuf, out_hbm.at[s, :, pl.ds(b0, blk)], osem)

        def body(g, carry):
            s = g * 2
            phase(s, s + 1 < seq, gbuf0, gsem0, gbuf1, gsem1, tbuf0, osem0,
                  g > 0)
            phase(s + 1, s + 2 < seq, gbuf1, gsem1, gbuf0, gsem0, tbuf1, osem1,
                  g > 0)
            return carry

        lax.fori_loop(0, seq // 2, body, 0)
        pltpu.make_async_copy(tbuf0, out_hbm.at[0, :, pl.ds(b0, blk)], osem0).wait()
        pltpu.make_async_copy(tbuf1, out_hbm.at[0, :, pl.ds(b0, blk)], osem1).wait()

    return k


def kernel(x, embs):
    b, s = x.shape
    v, d = embs.shape
    idx_t = x.T.astype(jnp.int32)
    table = _make_table_pair(v, d, 4096)(embs.T)
    out_t = _make_lookup(s, b, d, 2 * d, 128)(idx_t, table)
    return jnp.transpose(out_t, (2, 0, 1))
